# baseline (speedup 1.0000x reference)
import functools

import jax
import jax.numpy as jnp
from jax import lax
from jax.experimental import pallas as pl
from jax.experimental.pallas import tpu as pltpu
from jax.experimental.pallas import tpu_sc as plsc

_N = 10000
_E = 320000
_DIN = 128
_HID = 64
_NCLS = 10
_NG = 32
_EPS = 1e-5

_NC = 2
_NS = 16
_NW = _NC * _NS
_EPW = _E // _NW
_CHUNK = 80
_NCHUNK = _EPW // _CHUNK
_RPT = 624
_REM = _N - _RPT * _NS
_REMOFF = _RPT * _NS
_DEGW = 128

_mesh = plsc.VectorSubcoreMesh(core_axis_name="c", subcore_axis_name="s")


@functools.partial(
    pl.kernel,
    mesh=_mesh,
    out_type=jax.ShapeDtypeStruct((_NC, _N, _DEGW), jnp.float32),
    scratch_types=[
        pltpu.VMEM((_NCHUNK, _CHUNK), jnp.int32),
        pltpu.VMEM((_CHUNK, _DEGW), jnp.float32),
        pltpu.VMEM_SHARED((_N, _DEGW), jnp.float32),
    ],
)
def _deg_scatter(dst_hbm, ones_hbm, zeros_hbm, out_hbm, dst_v, ones_v, acc_sh):
    cid = lax.axis_index("c")
    sid = lax.axis_index("s")
    wid = sid * _NC + cid
    _init_rows(zeros_hbm, acc_sh, sid)
    pltpu.sync_copy(ones_hbm, ones_v)
    pltpu.sync_copy(dst_hbm.at[wid], dst_v)
    plsc.subcore_barrier()

    def body(i, carry):
        pltpu.sync_copy(ones_v, acc_sh.at[dst_v.at[i]], add=True)
        return carry

    lax.fori_loop(0, _NCHUNK, body, 0)
    plsc.subcore_barrier()
    _write_rows(acc_sh, out_hbm, cid, sid)


def _init_rows(zeros_hbm, acc_sh, sid):
    pltpu.sync_copy(zeros_hbm.at[pl.ds(sid * _RPT, _RPT)],
                    acc_sh.at[pl.ds(sid * _RPT, _RPT)])

    @pl.when(sid == _NS - 1)
    def _():
        pltpu.sync_copy(zeros_hbm.at[pl.ds(_REMOFF, _REM)],
                        acc_sh.at[pl.ds(_REMOFF, _REM)])


def _write_rows(acc_sh, out_hbm, cid, sid):
    pltpu.sync_copy(acc_sh.at[pl.ds(sid * _RPT, _RPT)],
                    out_hbm.at[cid, pl.ds(sid * _RPT, _RPT)])

    @pl.when(sid == _NS - 1)
    def _():
        pltpu.sync_copy(acc_sh.at[pl.ds(_REMOFF, _REM)],
                        out_hbm.at[cid, pl.ds(_REMOFF, _REM)])


def _make_edge(D):
    @functools.partial(
        pl.kernel,
        mesh=_mesh,
        out_type=jax.ShapeDtypeStruct((_NC, _N, D), jnp.float32),
        scratch_types=[
            pltpu.VMEM((_NCHUNK, _CHUNK), jnp.int32),
            pltpu.VMEM((_CHUNK,), jnp.int32),
            pltpu.VMEM((_CHUNK,), jnp.int32),
            pltpu.VMEM((_CHUNK, D), jnp.float32),
            pltpu.VMEM((_CHUNK, D), jnp.float32),
            pltpu.VMEM_SHARED((_N, D), jnp.float32),
            pltpu.SemaphoreType.DMA,
            pltpu.SemaphoreType.DMA,
            pltpu.SemaphoreType.DMA,
            pltpu.SemaphoreType.DMA,
            pltpu.SemaphoreType.DMA,
            pltpu.SemaphoreType.DMA,
        ],
    )
    def edge_k(z_hbm, src_hbm, dst_hbm, zeros_hbm, out_hbm,
               dst_v, si0_v, si1_v, rows0_v, rows1_v, acc_sh,
               sf0, sf1, sg0, sg1, ss0, ss1):
        cid = lax.axis_index("c")
        sid = lax.axis_index("s")
        wid = sid * _NC + cid
        _init_rows(zeros_hbm, acc_sh, sid)
        pltpu.sync_copy(dst_hbm.at[wid], dst_v)
        plsc.subcore_barrier()

        sib = (si0_v, si1_v)
        rows = (rows0_v, rows1_v)
        sf = (sf0, sf1)
        sg = (sg0, sg1)
        ss = (ss0, ss1)
        base = wid * _EPW

        def fetch(i, b):
            return pltpu.make_async_copy(
                src_hbm.at[pl.ds(base + i * _CHUNK, _CHUNK)], sib[b], sf[b])

        def gather(b):
            return pltpu.make_async_copy(z_hbm.at[sib[b]], rows[b], sg[b])

        class _Scatter:
            def __init__(self, i, b):
                self.i, self.b = i, b

            def start(self):
                pltpu.async_copy(rows[self.b], acc_sh.at[dst_v.at[self.i]],
                                 ss[self.b], add=True)

            def wait(self):
                pltpu.make_async_copy(rows[self.b],
                                      acc_sh.at[dst_v.at[self.i]],
                                      ss[self.b]).wait()

        scatter = _Scatter

        last = _NCHUNK - 1

        fetch(0, 0).start()
        fetch(0, 0).wait()
        gather(0).start()
        fetch(1, 1).start()
        gather(0).wait()
        scatter(0, 0).start()
        fetch(1, 1).wait()
        gather(1).start()
        fetch(2, 0).start()
        gather(1).wait()
        scatter(1, 1).start()
        fetch(2, 0).wait()
        scatter(0, 0).wait()
        gather(0).start()
        fetch(3, 1).start()

        def body(i2, carry):
            for b in range(2):
                i = 2 * i2 + b
                gather(b).wait()
                scatter(i - 1, 1 - b).wait()
                scatter(i, b).start()
                fetch(i + 1, 1 - b).wait()
                gather(1 - b).start()

                @pl.when(i + 2 <= last)
                def _():
                    fetch(i + 2, b).start()
            return carry

        lax.fori_loop(1, (_NCHUNK - 1) // 2, body, 0)
        gather(last % 2).wait()
        scatter(last - 1, 1 - last % 2).wait()
        scatter(last, last % 2).start()
        scatter(last, last % 2).wait()
        plsc.subcore_barrier()
        _write_rows(acc_sh, out_hbm, cid, sid)

    return edge_k


_edge128 = _make_edge(2 * _HID)


def _dinv(degp_ref):
    deg = degp_ref[0, :, 0:1] + degp_ref[1, :, 0:1] + 1.0
    return lax.rsqrt(deg)


def _z1_body(x_ref, w_ref, degp_ref, out_ref):
    out_ref[...] = jnp.dot(x_ref[...], w_ref[...],
                           preferred_element_type=jnp.float32) * _dinv(degp_ref)


def _mid_body(sp_ref, z_ref, degp_ref, b_ref, g_ref, be_ref, w2_ref, out_ref):
    dinv = _dinv(degp_ref)
    y = (sp_ref[0] + sp_ref[1] + z_ref[...]) * dinv + b_ref[...]
    m = jnp.mean(y, axis=0, keepdims=True)
    d = y - m
    v = jnp.mean(d * d, axis=0, keepdims=True)
    h = jnp.maximum(g_ref[...] * d * lax.rsqrt(v + _EPS) + be_ref[...], 0.0)
    out_ref[...] = jnp.dot(h, w2_ref[...],
                           preferred_element_type=jnp.float32) * dinv


def _final_body(sp_ref, z_ref, degp_ref, b_ref, g_ref, be_ref, batch_ref,
                wn_ref, bn_ref, wf_ref, bf_ref, out_ref):
    dinv = _dinv(degp_ref)
    y = (sp_ref[0] + sp_ref[1] + z_ref[...]) * dinv + b_ref[...]
    m = jnp.mean(y, axis=0, keepdims=True)
    d = y - m
    v = jnp.mean(d * d, axis=0, keepdims=True)
    h = jnp.maximum(g_ref[...] * d * lax.rsqrt(v + _EPS) + be_ref[...], 0.0)
    gids = lax.broadcasted_iota(jnp.int32, (_NG, _N), 0)
    onehot = jnp.where(gids == batch_ref[...], 1.0, 0.0)
    pooled = jnp.dot(onehot, h, preferred_element_type=jnp.float32)
    p = jnp.dot(pooled, wn_ref[...], preferred_element_type=jnp.float32) + bn_ref[...]
    logits = jnp.dot(p, wf_ref[...], preferred_element_type=jnp.float32) + bf_ref[...]
    mx = jnp.max(logits, axis=1, keepdims=True)
    e = jnp.exp(logits - mx)
    out_ref[...] = (logits - mx) - jnp.log(jnp.sum(e, axis=1, keepdims=True))


def kernel(x, edge_index, batch, image_features, W1, b1, g1, be1,
           W2, b2, g2, be2, Wn, bn, Wf, bf):
    del image_features
    edge_index = edge_index.reshape(2, -1)
    src = edge_index[0]
    dst = edge_index[1].reshape(_NW, _NCHUNK, _CHUNK)

    deg_p = _deg_scatter(dst,
                         jnp.ones((_CHUNK, _DEGW), jnp.float32),
                         jnp.zeros((_N, _DEGW), jnp.float32))

    z1 = pl.pallas_call(
        _z1_body,
        out_shape=jax.ShapeDtypeStruct((_N, 2 * _HID), jnp.float32),
    )(x, W1, deg_p)

    zeros128 = jnp.zeros((_N, 2 * _HID), jnp.float32)
    s1 = _edge128(z1, src, dst, zeros128)

    pad = 2 * _HID - _HID
    w2p = jnp.pad(W2, ((0, 0), (0, pad)))
    z2 = pl.pallas_call(
        _mid_body,
        out_shape=jax.ShapeDtypeStruct((_N, 2 * _HID), jnp.float32),
    )(s1, z1, deg_p, b1.reshape(1, -1), g1.reshape(1, -1),
      be1.reshape(1, -1), w2p)

    s2 = _edge128(z2, src, dst, zeros128)

    out = pl.pallas_call(
        _final_body,
        out_shape=jax.ShapeDtypeStruct((_NG, _NCLS), jnp.float32),
    )(s2, z2, deg_p,
      jnp.pad(b2, (0, pad)).reshape(1, -1),
      jnp.pad(g2, (0, pad)).reshape(1, -1),
      jnp.pad(be2, (0, pad)).reshape(1, -1),
      batch.reshape(1, -1),
      jnp.pad(Wn, ((0, pad), (0, 0))), bn.reshape(1, -1),
      Wf, bf.reshape(1, -1))
    return out


# speedup vs baseline: 1.0032x; 1.0032x over previous
import functools

import jax
import jax.numpy as jnp
from jax import lax
from jax.experimental import pallas as pl
from jax.experimental.pallas import tpu as pltpu
from jax.experimental.pallas import tpu_sc as plsc

_N = 10000
_E = 320000
_DIN = 128
_HID = 64
_NCLS = 10
_NG = 32
_EPS = 1e-5

_NC = 2
_NS = 16
_NW = _NC * _NS
_EPW = _E // _NW
_CHUNK = 80
_NCHUNK = _EPW // _CHUNK
_RPT = 624
_REM = _N - _RPT * _NS
_REMOFF = _RPT * _NS
_DEGW = 128

_mesh = plsc.VectorSubcoreMesh(core_axis_name="c", subcore_axis_name="s")


@functools.partial(
    pl.kernel,
    mesh=_mesh,
    out_type=jax.ShapeDtypeStruct((_NC, _N, _DEGW), jnp.float32),
    scratch_types=[
        pltpu.VMEM((_NCHUNK, _CHUNK), jnp.int32),
        pltpu.VMEM((_CHUNK, _DEGW), jnp.float32),
        pltpu.VMEM_SHARED((_N, _DEGW), jnp.float32),
        pltpu.SemaphoreType.DMA,
        pltpu.SemaphoreType.DMA,
    ],
)
def _deg_scatter(dst_hbm, ones_hbm, zeros_hbm, out_hbm, dst_v, ones_v, acc_sh,
                 ss0, ss1):
    cid = lax.axis_index("c")
    sid = lax.axis_index("s")
    wid = sid * _NC + cid
    _init_rows(zeros_hbm, acc_sh, sid)
    pltpu.sync_copy(ones_hbm, ones_v)
    pltpu.sync_copy(dst_hbm.at[wid], dst_v)
    plsc.subcore_barrier()
    ss = (ss0, ss1)

    def start(i, b):
        pltpu.async_copy(ones_v, acc_sh.at[dst_v.at[i]], ss[b], add=True)

    def wait(i, b):
        pltpu.make_async_copy(ones_v, acc_sh.at[dst_v.at[i]], ss[b]).wait()

    start(0, 0)
    start(1, 1)

    def body(i2, carry):
        for b in range(2):
            i = 2 * i2 + b
            wait(i - 2, b)
            start(i, b)
        return carry

    lax.fori_loop(1, (_NCHUNK - 1) // 2, body, 0)
    last = _NCHUNK - 1
    wait(last - 2, 0)
    start(last, 0)
    wait(last - 1, 1)
    wait(last, 0)
    plsc.subcore_barrier()
    _write_rows(acc_sh, out_hbm, cid, sid)


def _init_rows(zeros_hbm, acc_sh, sid):
    pltpu.sync_copy(zeros_hbm.at[pl.ds(sid * _RPT, _RPT)],
                    acc_sh.at[pl.ds(sid * _RPT, _RPT)])

    @pl.when(sid == _NS - 1)
    def _():
        pltpu.sync_copy(zeros_hbm.at[pl.ds(_REMOFF, _REM)],
                        acc_sh.at[pl.ds(_REMOFF, _REM)])


def _write_rows(acc_sh, out_hbm, cid, sid):
    pltpu.sync_copy(acc_sh.at[pl.ds(sid * _RPT, _RPT)],
                    out_hbm.at[cid, pl.ds(sid * _RPT, _RPT)])

    @pl.when(sid == _NS - 1)
    def _():
        pltpu.sync_copy(acc_sh.at[pl.ds(_REMOFF, _REM)],
                        out_hbm.at[cid, pl.ds(_REMOFF, _REM)])


def _make_edge(D):
    @functools.partial(
        pl.kernel,
        mesh=_mesh,
        out_type=jax.ShapeDtypeStruct((_NC, _N, D), jnp.float32),
        scratch_types=[
            pltpu.VMEM((_NCHUNK, _CHUNK), jnp.int32),
            pltpu.VMEM((_CHUNK,), jnp.int32),
            pltpu.VMEM((_CHUNK,), jnp.int32),
            pltpu.VMEM((_CHUNK, D), jnp.float32),
            pltpu.VMEM((_CHUNK, D), jnp.float32),
            pltpu.VMEM_SHARED((_N, D), jnp.float32),
            pltpu.SemaphoreType.DMA,
            pltpu.SemaphoreType.DMA,
            pltpu.SemaphoreType.DMA,
            pltpu.SemaphoreType.DMA,
            pltpu.SemaphoreType.DMA,
            pltpu.SemaphoreType.DMA,
        ],
    )
    def edge_k(z_hbm, src_hbm, dst_hbm, zeros_hbm, out_hbm,
               dst_v, si0_v, si1_v, rows0_v, rows1_v, acc_sh,
               sf0, sf1, sg0, sg1, ss0, ss1):
        cid = lax.axis_index("c")
        sid = lax.axis_index("s")
        wid = sid * _NC + cid
        _init_rows(zeros_hbm, acc_sh, sid)
        pltpu.sync_copy(dst_hbm.at[wid], dst_v)
        plsc.subcore_barrier()

        sib = (si0_v, si1_v)
        rows = (rows0_v, rows1_v)
        sf = (sf0, sf1)
        sg = (sg0, sg1)
        ss = (ss0, ss1)
        base = wid * _EPW

        def fetch(i, b):
            return pltpu.make_async_copy(
                src_hbm.at[pl.ds(base + i * _CHUNK, _CHUNK)], sib[b], sf[b])

        def gather(b):
            return pltpu.make_async_copy(z_hbm.at[sib[b]], rows[b], sg[b])

        class _Scatter:
            def __init__(self, i, b):
                self.i, self.b = i, b

            def start(self):
                pltpu.async_copy(rows[self.b], acc_sh.at[dst_v.at[self.i]],
                                 ss[self.b], add=True)

            def wait(self):
                pltpu.make_async_copy(rows[self.b],
                                      acc_sh.at[dst_v.at[self.i]],
                                      ss[self.b]).wait()

        scatter = _Scatter

        last = _NCHUNK - 1

        fetch(0, 0).start()
        fetch(0, 0).wait()
        gather(0).start()
        fetch(1, 1).start()
        gather(0).wait()
        scatter(0, 0).start()
        fetch(1, 1).wait()
        gather(1).start()
        fetch(2, 0).start()
        gather(1).wait()
        scatter(1, 1).start()
        fetch(2, 0).wait()
        scatter(0, 0).wait()
        gather(0).start()
        fetch(3, 1).start()

        def body(i2, carry):
            for b in range(2):
                i = 2 * i2 + b
                gather(b).wait()
                scatter(i - 1, 1 - b).wait()
                scatter(i, b).start()
                fetch(i + 1, 1 - b).wait()
                gather(1 - b).start()

                @pl.when(i + 2 <= last)
                def _():
                    fetch(i + 2, b).start()
            return carry

        lax.fori_loop(1, (_NCHUNK - 1) // 2, body, 0)
        gather(last % 2).wait()
        scatter(last - 1, 1 - last % 2).wait()
        scatter(last, last % 2).start()
        scatter(last, last % 2).wait()
        plsc.subcore_barrier()
        _write_rows(acc_sh, out_hbm, cid, sid)

    return edge_k


_edge128 = _make_edge(2 * _HID)


def _dinv(degp_ref):
    deg = degp_ref[0, :, 0:1] + degp_ref[1, :, 0:1] + 1.0
    return lax.rsqrt(deg)


def _z1_body(x_ref, w_ref, degp_ref, out_ref):
    out_ref[...] = jnp.dot(x_ref[...], w_ref[...],
                           preferred_element_type=jnp.float32) * _dinv(degp_ref)


def _mid_body(sp_ref, z_ref, degp_ref, b_ref, g_ref, be_ref, w2_ref, out_ref):
    dinv = _dinv(degp_ref)
    y = (sp_ref[0] + sp_ref[1] + z_ref[...]) * dinv + b_ref[...]
    m = jnp.mean(y, axis=0, keepdims=True)
    d = y - m
    v = jnp.mean(d * d, axis=0, keepdims=True)
    h = jnp.maximum(g_ref[...] * d * lax.rsqrt(v + _EPS) + be_ref[...], 0.0)
    out_ref[...] = jnp.dot(h, w2_ref[...],
                           preferred_element_type=jnp.float32) * dinv


def _final_body(sp_ref, z_ref, degp_ref, b_ref, g_ref, be_ref, batch_ref,
                wn_ref, bn_ref, wf_ref, bf_ref, out_ref):
    dinv = _dinv(degp_ref)
    y = (sp_ref[0] + sp_ref[1] + z_ref[...]) * dinv + b_ref[...]
    m = jnp.mean(y, axis=0, keepdims=True)
    d = y - m
    v = jnp.mean(d * d, axis=0, keepdims=True)
    h = jnp.maximum(g_ref[...] * d * lax.rsqrt(v + _EPS) + be_ref[...], 0.0)
    gids = lax.broadcasted_iota(jnp.int32, (_NG, _N), 0)
    onehot = jnp.where(gids == batch_ref[...], 1.0, 0.0)
    pooled = jnp.dot(onehot, h, preferred_element_type=jnp.float32)
    p = jnp.dot(pooled, wn_ref[...], preferred_element_type=jnp.float32) + bn_ref[...]
    logits = jnp.dot(p, wf_ref[...], preferred_element_type=jnp.float32) + bf_ref[...]
    mx = jnp.max(logits, axis=1, keepdims=True)
    e = jnp.exp(logits - mx)
    out_ref[...] = (logits - mx) - jnp.log(jnp.sum(e, axis=1, keepdims=True))


def kernel(x, edge_index, batch, image_features, W1, b1, g1, be1,
           W2, b2, g2, be2, Wn, bn, Wf, bf):
    del image_features
    edge_index = edge_index.reshape(2, -1)
    src = edge_index[0]
    dst = edge_index[1].reshape(_NW, _NCHUNK, _CHUNK)

    deg_p = _deg_scatter(dst,
                         jnp.ones((_CHUNK, _DEGW), jnp.float32),
                         jnp.zeros((_N, _DEGW), jnp.float32))

    z1 = pl.pallas_call(
        _z1_body,
        out_shape=jax.ShapeDtypeStruct((_N, 2 * _HID), jnp.float32),
    )(x, W1, deg_p)

    zeros128 = jnp.zeros((_N, 2 * _HID), jnp.float32)
    s1 = _edge128(z1, src, dst, zeros128)

    pad = 2 * _HID - _HID
    w2p = jnp.pad(W2, ((0, 0), (0, pad)))
    z2 = pl.pallas_call(
        _mid_body,
        out_shape=jax.ShapeDtypeStruct((_N, 2 * _HID), jnp.float32),
    )(s1, z1, deg_p, b1.reshape(1, -1), g1.reshape(1, -1),
      be1.reshape(1, -1), w2p)

    s2 = _edge128(z2, src, dst, zeros128)

    out = pl.pallas_call(
        _final_body,
        out_shape=jax.ShapeDtypeStruct((_NG, _NCLS), jnp.float32),
    )(s2, z2, deg_p,
      jnp.pad(b2, (0, pad)).reshape(1, -1),
      jnp.pad(g2, (0, pad)).reshape(1, -1),
      jnp.pad(be2, (0, pad)).reshape(1, -1),
      batch.reshape(1, -1),
      jnp.pad(Wn, ((0, pad), (0, 0))), bn.reshape(1, -1),
      Wf, bf.reshape(1, -1))
    return out
